# baseline (device time: 496033 ns/iter reference)
import jax
import jax.numpy as jnp
from jax import lax
from jax.experimental import pallas as pl
from jax.experimental.pallas import tpu as pltpu

M = 4096
D = 4096
B = 256
NB = M // B


def kernel(partial, resid, gamma):
    pb = partial.reshape(M, D).astype(jnp.bfloat16)

    def body(pb_ref, r_ref, g_ref, out_ref, comm, send_sems, recv_sems):
        k = pl.program_id(0)
        my_x = lax.axis_index("x")
        my_y = lax.axis_index("y")
        my_z = lax.axis_index("z")
        nbr = (my_x, 1 - my_y, my_z)

        barrier_sem = pltpu.get_barrier_semaphore()

        @pl.when(k == 0)
        def _():
            pl.semaphore_signal(
                barrier_sem, inc=1,
                device_id=nbr, device_id_type=pl.DeviceIdType.MESH,
            )
            pl.semaphore_wait(barrier_sem, 1)

        slot = lax.rem(k, 4)
        rdma = pltpu.make_async_remote_copy(
            src_ref=pb_ref,
            dst_ref=comm.at[slot],
            send_sem=send_sems.at[slot],
            recv_sem=recv_sems.at[slot],
            device_id=nbr,
            device_id_type=pl.DeviceIdType.MESH,
        )
        rdma.start()
        rdma.wait()

        y = (
            pb_ref[...].astype(jnp.float32)
            + comm[slot].astype(jnp.float32)
            + r_ref[...]
        )
        ms = jnp.mean(y * y, axis=-1, keepdims=True)
        out_ref[...] = y * lax.rsqrt(ms + 1e-6) * g_ref[...][None, :]

    return pl.pallas_call(
        body,
        grid=(NB,),
        in_specs=[
            pl.BlockSpec((B, D), lambda i: (i, 0)),
            pl.BlockSpec((B, D), lambda i: (i, 0)),
            pl.BlockSpec((D,), lambda i: (0,)),
        ],
        out_specs=pl.BlockSpec((B, D), lambda i: (i, 0)),
        out_shape=jax.ShapeDtypeStruct((M, D), jnp.float32),
        scratch_shapes=[
            pltpu.VMEM((4, B, D), jnp.bfloat16),
            pltpu.SemaphoreType.DMA((4,)),
            pltpu.SemaphoreType.DMA((4,)),
        ],
        compiler_params=pltpu.CompilerParams(collective_id=0),
    )(pb, resid, gamma)


# device time: 326856 ns/iter; 1.5176x vs baseline; 1.5176x over previous
import jax
import jax.numpy as jnp
from jax import lax
from jax.experimental import pallas as pl
from jax.experimental.pallas import tpu as pltpu

M = 4096
D = 4096
HALF = M // 2
B = 256
NBH = HALF // B
S = 4
L = 2


def kernel(partial, resid, gamma):
    pb = partial.reshape(M, D).astype(jnp.bfloat16)
    rb = resid.astype(jnp.bfloat16)

    def body(pb_ref, rb_ref, g_ref, out_ref,
             ohA, ohB, pbA, pbB, ridA, ridB, outA, outB,
             sy_send, sy_recv, sx_send, sx_recv,
             s_pbA, s_pbB, s_ridA, s_ridB, s_outA, s_outB):
        x = lax.axis_index("x")
        y = lax.axis_index("y")
        z = lax.axis_index("z")
        ynbr = (x, 1 - y, z)
        xnbr = (1 - x, y, z)
        baseA = x * HALF
        baseB = (1 - x) * HALF

        def ysend(j):
            return pltpu.make_async_remote_copy(
                src_ref=pbA.at[j % S],
                dst_ref=ohA.at[j % S],
                send_sem=sy_send.at[j % S],
                recv_sem=sy_recv.at[j % S],
                device_id=ynbr,
                device_id_type=pl.DeviceIdType.MESH,
            )

        def xfwd(j):
            return pltpu.make_async_remote_copy(
                src_ref=ohA.at[j % S],
                dst_ref=ohB.at[j % S],
                send_sem=sx_send.at[j % S],
                recv_sem=sx_recv.at[j % S],
                device_id=xnbr,
                device_id_type=pl.DeviceIdType.MESH,
            )

        def load(hbm, row, dst, sem, wait=False):
            cp = pltpu.make_async_copy(hbm.at[pl.ds(row, B), :], dst, sem)
            cp.wait() if wait else cp.start()

        def store(src, row, sem, wait=False):
            cp = pltpu.make_async_copy(src, out_ref.at[pl.ds(row, B), :], sem)
            cp.wait() if wait else cp.start()

        def rmsnorm(v):
            ms = jnp.mean(v * v, axis=-1, keepdims=True)
            return v * lax.rsqrt(ms + 1e-6) * g_ref[...][None, :]

        bsem = pltpu.get_barrier_semaphore()
        for nbr in (ynbr, xnbr):
            pl.semaphore_signal(
                bsem, inc=1, device_id=nbr,
                device_id_type=pl.DeviceIdType.MESH,
            )
        pl.semaphore_wait(bsem, 2)

        load(pb_ref, baseA, pbA.at[0], s_pbA.at[0])
        load(rb_ref, baseA, ridA.at[0], s_ridA.at[0])

        for k in range(NBH + 2):
            jw = k - S + 1
            if 0 <= jw < NBH:
                ysend(jw).wait_send()

            if k < NBH:
                load(pb_ref, baseA + k * B, pbA.at[k % S], s_pbA.at[k % S],
                     wait=True)
                ysend(k).start()

            j = k - 1
            if 0 <= j < NBH:
                ysend(j).wait_recv()
                jf = j - S
                if 0 <= jf < NBH:
                    xfwd(jf).wait_send()
                xfwd(j).start()
                load(rb_ref, baseA + j * B, ridA.at[j % L], s_ridA.at[j % L],
                     wait=True)
                if j >= L:
                    store(outA.at[j % L], baseA + (j - L) * B,
                          s_outA.at[j % L], wait=True)
                vA = (pbA[j % S, :, :].astype(jnp.float32)
                      + ohA[j % S, :, :].astype(jnp.float32)
                      + ridA[j % L, :, :].astype(jnp.float32))
                outA[j % L, :, :] = rmsnorm(vA)
                store(outA.at[j % L], baseA + j * B, s_outA.at[j % L])

            j2 = k - 2
            if 0 <= j2 < NBH:
                xfwd(j2).wait_recv()
                load(pb_ref, baseB + j2 * B, pbB.at[j2 % L], s_pbB.at[j2 % L],
                     wait=True)
                load(rb_ref, baseB + j2 * B, ridB.at[j2 % L],
                     s_ridB.at[j2 % L], wait=True)
                if j2 >= L:
                    store(outB.at[j2 % L], baseB + (j2 - L) * B,
                          s_outB.at[j2 % L], wait=True)
                vB = (pbB[j2 % L, :, :].astype(jnp.float32)
                      + ohB[j2 % S, :, :].astype(jnp.float32)
                      + ridB[j2 % L, :, :].astype(jnp.float32))
                outB[j2 % L, :, :] = rmsnorm(vB)
                store(outB.at[j2 % L], baseB + j2 * B, s_outB.at[j2 % L])

            if k + 1 < NBH:
                load(pb_ref, baseA + (k + 1) * B, pbA.at[(k + 1) % S],
                     s_pbA.at[(k + 1) % S])
                load(rb_ref, baseA + (k + 1) * B, ridA.at[(k + 1) % L],
                     s_ridA.at[(k + 1) % L])
            if k < NBH:
                load(pb_ref, baseB + k * B, pbB.at[k % L], s_pbB.at[k % L])
                load(rb_ref, baseB + k * B, ridB.at[k % L], s_ridB.at[k % L])

        ysend(NBH - 1).wait_send()
        for j in range(max(0, NBH - S), NBH):
            xfwd(j).wait_send()
        for j in range(max(0, NBH - L), NBH):
            store(outA.at[j % L], baseA + j * B, s_outA.at[j % L], wait=True)
            store(outB.at[j % L], baseB + j * B, s_outB.at[j % L], wait=True)

    return pl.pallas_call(
        body,
        in_specs=[
            pl.BlockSpec(memory_space=pl.ANY),
            pl.BlockSpec(memory_space=pl.ANY),
            pl.BlockSpec(memory_space=pltpu.VMEM),
        ],
        out_specs=pl.BlockSpec(memory_space=pl.ANY),
        out_shape=jax.ShapeDtypeStruct((M, D), jnp.float32),
        scratch_shapes=[
            pltpu.VMEM((S, B, D), jnp.bfloat16),
            pltpu.VMEM((S, B, D), jnp.bfloat16),
            pltpu.VMEM((S, B, D), jnp.bfloat16),
            pltpu.VMEM((L, B, D), jnp.bfloat16),
            pltpu.VMEM((L, B, D), jnp.bfloat16),
            pltpu.VMEM((L, B, D), jnp.bfloat16),
            pltpu.VMEM((L, B, D), jnp.float32),
            pltpu.VMEM((L, B, D), jnp.float32),
            pltpu.SemaphoreType.DMA((S,)),
            pltpu.SemaphoreType.DMA((S,)),
            pltpu.SemaphoreType.DMA((S,)),
            pltpu.SemaphoreType.DMA((S,)),
            pltpu.SemaphoreType.DMA((S,)),
            pltpu.SemaphoreType.DMA((L,)),
            pltpu.SemaphoreType.DMA((L,)),
            pltpu.SemaphoreType.DMA((L,)),
            pltpu.SemaphoreType.DMA((L,)),
            pltpu.SemaphoreType.DMA((L,)),
        ],
        compiler_params=pltpu.CompilerParams(
            collective_id=0, vmem_limit_bytes=60 * 1024 * 1024,
        ),
    )(pb, rb, gamma)


# device time: 261337 ns/iter; 1.8981x vs baseline; 1.2507x over previous
import jax
import jax.numpy as jnp
from jax import lax
from jax.experimental import pallas as pl
from jax.experimental.pallas import tpu as pltpu

M = 4096
D = 4096
HALF = M // 2
B = 256
NBH = HALF // B
R = 4
T = 3


def kernel(partial, resid, gamma):
    pb = partial.reshape(M, D)

    def body(pb_ref, rb_ref, g_ref, out_ref,
             ohA, ohB, pbA, pbF, ridA, fwd, outA, outB,
             sy_send, sy_recv, sx_send, sx_recv,
             s_pbF, s_ridA, s_outA, s_outB):
        x = lax.axis_index("x")
        y = lax.axis_index("y")
        z = lax.axis_index("z")
        ynbr = (x, 1 - y, z)
        xnbr = (1 - x, y, z)
        baseA = x * HALF
        baseB = (1 - x) * HALF

        def ysend(j):
            return pltpu.make_async_remote_copy(
                src_ref=pbA.at[j % T],
                dst_ref=ohA.at[j % R],
                send_sem=sy_send.at[j % T],
                recv_sem=sy_recv.at[j % R],
                device_id=ynbr,
                device_id_type=pl.DeviceIdType.MESH,
            )

        def xfwd(j):
            return pltpu.make_async_remote_copy(
                src_ref=fwd.at[j % T],
                dst_ref=ohB.at[j % R],
                send_sem=sx_send.at[j % T],
                recv_sem=sx_recv.at[j % R],
                device_id=xnbr,
                device_id_type=pl.DeviceIdType.MESH,
            )

        def load(hbm, row, dst, sem, wait=False):
            cp = pltpu.make_async_copy(hbm.at[pl.ds(row, B), :], dst, sem)
            cp.wait() if wait else cp.start()

        def store(src, row, sem, wait=False):
            cp = pltpu.make_async_copy(src, out_ref.at[pl.ds(row, B), :], sem)
            cp.wait() if wait else cp.start()

        def norm(v):
            ms = jnp.mean(v * v, axis=-1, keepdims=True)
            return v * lax.rsqrt(ms + 1e-6) * g_ref[...][None, :]

        bsem = pltpu.get_barrier_semaphore()
        for nbr in (ynbr, xnbr):
            pl.semaphore_signal(
                bsem, inc=1, device_id=nbr,
                device_id_type=pl.DeviceIdType.MESH,
            )
        pl.semaphore_wait(bsem, 2)

        load(pb_ref, baseA, pbF.at[0], s_pbF.at[0])
        load(rb_ref, baseA, ridA.at[0], s_ridA.at[0])
        if NBH > 1:
            load(rb_ref, baseA + B, ridA.at[1], s_ridA.at[1])

        for k in range(NBH + 2):
            if 0 <= k - T < NBH:
                ysend(k - T).wait_send()

            if k < NBH:
                load(pb_ref, baseA + k * B, pbF.at[k % 2], s_pbF.at[k % 2],
                     wait=True)
                pbA[k % T, :, :] = pbF[k % 2, :, :].astype(jnp.bfloat16)
                ysend(k).start()

            if k + 1 < NBH:
                load(pb_ref, baseA + (k + 1) * B, pbF.at[(k + 1) % 2],
                     s_pbF.at[(k + 1) % 2])

            j = k - 1
            if 0 <= j < NBH:
                ysend(j).wait_recv()
                if 0 <= j - T < NBH:
                    xfwd(j - T).wait_send()
                load(rb_ref, baseA + j * B, ridA.at[j % 2], s_ridA.at[j % 2],
                     wait=True)
                yA = (pbA[j % T, :, :].astype(jnp.float32)
                      + ohA[j % R, :, :].astype(jnp.float32)
                      + ridA[j % 2, :, :])
                fwd[j % T, :, :] = yA.astype(jnp.bfloat16)
                xfwd(j).start()
                if j >= 1:
                    store(outA, baseA + (j - 1) * B, s_outA, wait=True)
                outA[...] = norm(yA)
                store(outA, baseA + j * B, s_outA)
                if j + 2 < NBH:
                    load(rb_ref, baseA + (j + 2) * B, ridA.at[j % 2],
                         s_ridA.at[j % 2])

            j2 = k - 2
            if 0 <= j2 < NBH:
                xfwd(j2).wait_recv()
                if j2 >= 1:
                    store(outB, baseB + (j2 - 1) * B, s_outB, wait=True)
                outB[...] = norm(ohB[j2 % R, :, :].astype(jnp.float32))
                store(outB, baseB + j2 * B, s_outB)

        ysend(NBH - 1).wait_send()
        for j in range(max(0, NBH - T), NBH):
            xfwd(j).wait_send()
        store(outA, baseA + (NBH - 1) * B, s_outA, wait=True)
        store(outB, baseB + (NBH - 1) * B, s_outB, wait=True)

    return pl.pallas_call(
        body,
        in_specs=[
            pl.BlockSpec(memory_space=pl.ANY),
            pl.BlockSpec(memory_space=pl.ANY),
            pl.BlockSpec(memory_space=pltpu.VMEM),
        ],
        out_specs=pl.BlockSpec(memory_space=pl.ANY),
        out_shape=jax.ShapeDtypeStruct((M, D), jnp.float32),
        scratch_shapes=[
            pltpu.VMEM((R, B, D), jnp.bfloat16),
            pltpu.VMEM((R, B, D), jnp.bfloat16),
            pltpu.VMEM((T, B, D), jnp.bfloat16),
            pltpu.VMEM((2, B, D), jnp.float32),
            pltpu.VMEM((2, B, D), jnp.float32),
            pltpu.VMEM((T, B, D), jnp.bfloat16),
            pltpu.VMEM((B, D), jnp.float32),
            pltpu.VMEM((B, D), jnp.float32),
            pltpu.SemaphoreType.DMA((T,)),
            pltpu.SemaphoreType.DMA((R,)),
            pltpu.SemaphoreType.DMA((T,)),
            pltpu.SemaphoreType.DMA((R,)),
            pltpu.SemaphoreType.DMA((2,)),
            pltpu.SemaphoreType.DMA((2,)),
            pltpu.SemaphoreType.DMA,
            pltpu.SemaphoreType.DMA,
        ],
        compiler_params=pltpu.CompilerParams(
            collective_id=0, vmem_limit_bytes=60 * 1024 * 1024,
        ),
    )(pb, resid, gamma)


# device time: 260194 ns/iter; 1.9064x vs baseline; 1.0044x over previous
import jax
import jax.numpy as jnp
from jax import lax
from jax.experimental import pallas as pl
from jax.experimental.pallas import tpu as pltpu

M = 4096
D = 4096
HALF = M // 2
B = 256
NBH = HALF // B
R = 4
T = 3


def kernel(partial, resid, gamma):
    pb = partial.reshape(M, D)

    def body(pb_ref, rb_ref, g_ref, out_ref,
             ohA, ohB, pbA, pbF, ridA, fwd, outA, outB,
             sy_send, sy_recv, sx_send, sx_recv,
             s_pbF, s_ridA, s_outA, s_outB):
        x = lax.axis_index("x")
        y = lax.axis_index("y")
        z = lax.axis_index("z")
        ynbr = (x, 1 - y, z)
        xnbr = (1 - x, y, z)
        baseA = x * HALF
        baseB = (1 - x) * HALF

        def ysend(j):
            return pltpu.make_async_remote_copy(
                src_ref=pbA.at[j % T],
                dst_ref=ohA.at[j % R],
                send_sem=sy_send.at[j % T],
                recv_sem=sy_recv.at[j % R],
                device_id=ynbr,
                device_id_type=pl.DeviceIdType.MESH,
            )

        def xfwd(j):
            return pltpu.make_async_remote_copy(
                src_ref=fwd.at[j % T],
                dst_ref=ohB.at[j % R],
                send_sem=sx_send.at[j % T],
                recv_sem=sx_recv.at[j % R],
                device_id=xnbr,
                device_id_type=pl.DeviceIdType.MESH,
            )

        def load(hbm, row, dst, sem, wait=False):
            cp = pltpu.make_async_copy(hbm.at[pl.ds(row, B), :], dst, sem)
            cp.wait() if wait else cp.start()

        def store(src, row, sem, wait=False):
            cp = pltpu.make_async_copy(src, out_ref.at[pl.ds(row, B), :], sem)
            cp.wait() if wait else cp.start()

        def norm(v):
            ms = jnp.mean(v * v, axis=-1, keepdims=True)
            return v * lax.rsqrt(ms + 1e-6) * g_ref[...][None, :]

        bsem = pltpu.get_barrier_semaphore()
        for nbr in (ynbr, xnbr):
            pl.semaphore_signal(
                bsem, inc=1, device_id=nbr,
                device_id_type=pl.DeviceIdType.MESH,
            )
        pl.semaphore_wait(bsem, 2)

        load(pb_ref, baseA, pbF.at[0], s_pbF.at[0])
        load(rb_ref, baseA, ridA.at[0], s_ridA.at[0])
        if NBH > 1:
            load(rb_ref, baseA + B, ridA.at[1], s_ridA.at[1])
            load(pb_ref, baseA + B, pbF.at[1], s_pbF.at[1])
        load(pb_ref, baseA, pbF.at[0], s_pbF.at[0], wait=True)
        pbA[0, :, :] = pbF[0, :, :].astype(jnp.bfloat16)

        for k in range(NBH + 2):
            if k < NBH:
                ysend(k).start()

            if k + 2 < NBH:
                load(pb_ref, baseA + (k + 2) * B, pbF.at[k % 2],
                     s_pbF.at[k % 2])

            j = k - 1
            if 0 <= j < NBH:
                ysend(j).wait_recv()
                if 0 <= j - T < NBH:
                    xfwd(j - T).wait_send()
                load(rb_ref, baseA + j * B, ridA.at[j % 2], s_ridA.at[j % 2],
                     wait=True)
                yA = (pbA[j % T, :, :].astype(jnp.float32)
                      + ohA[j % R, :, :].astype(jnp.float32)
                      + ridA[j % 2, :, :])
                fwd[j % T, :, :] = yA.astype(jnp.bfloat16)
                xfwd(j).start()
                if j >= 1:
                    store(outA, baseA + (j - 1) * B, s_outA, wait=True)
                outA[...] = norm(yA)
                store(outA, baseA + j * B, s_outA)
                if j + 2 < NBH:
                    load(rb_ref, baseA + (j + 2) * B, ridA.at[j % 2],
                         s_ridA.at[j % 2])

            j2 = k - 2
            if 0 <= j2 < NBH:
                xfwd(j2).wait_recv()
                if j2 >= 1:
                    store(outB, baseB + (j2 - 1) * B, s_outB, wait=True)
                outB[...] = norm(ohB[j2 % R, :, :].astype(jnp.float32))
                store(outB, baseB + j2 * B, s_outB)

            if k + 1 < NBH:
                if 0 <= k + 1 - T < NBH:
                    ysend(k + 1 - T).wait_send()
                load(pb_ref, baseA + (k + 1) * B, pbF.at[(k + 1) % 2],
                     s_pbF.at[(k + 1) % 2], wait=True)
                pbA[(k + 1) % T, :, :] = (
                    pbF[(k + 1) % 2, :, :].astype(jnp.bfloat16))

        for j in range(max(0, NBH - T), NBH):
            ysend(j).wait_send()
        for j in range(max(0, NBH - T), NBH):
            xfwd(j).wait_send()
        store(outA, baseA + (NBH - 1) * B, s_outA, wait=True)
        store(outB, baseB + (NBH - 1) * B, s_outB, wait=True)

    return pl.pallas_call(
        body,
        in_specs=[
            pl.BlockSpec(memory_space=pl.ANY),
            pl.BlockSpec(memory_space=pl.ANY),
            pl.BlockSpec(memory_space=pltpu.VMEM),
        ],
        out_specs=pl.BlockSpec(memory_space=pl.ANY),
        out_shape=jax.ShapeDtypeStruct((M, D), jnp.float32),
        scratch_shapes=[
            pltpu.VMEM((R, B, D), jnp.bfloat16),
            pltpu.VMEM((R, B, D), jnp.bfloat16),
            pltpu.VMEM((T, B, D), jnp.bfloat16),
            pltpu.VMEM((2, B, D), jnp.float32),
            pltpu.VMEM((2, B, D), jnp.float32),
            pltpu.VMEM((T, B, D), jnp.bfloat16),
            pltpu.VMEM((B, D), jnp.float32),
            pltpu.VMEM((B, D), jnp.float32),
            pltpu.SemaphoreType.DMA((T,)),
            pltpu.SemaphoreType.DMA((R,)),
            pltpu.SemaphoreType.DMA((T,)),
            pltpu.SemaphoreType.DMA((R,)),
            pltpu.SemaphoreType.DMA((2,)),
            pltpu.SemaphoreType.DMA((2,)),
            pltpu.SemaphoreType.DMA,
            pltpu.SemaphoreType.DMA,
        ],
        compiler_params=pltpu.CompilerParams(
            collective_id=0, vmem_limit_bytes=60 * 1024 * 1024,
        ),
    )(pb, resid, gamma)
